# speedup vs baseline: 1.7007x; 1.7007x over previous
import jax
import jax.numpy as jnp
from jax import lax
from jax.experimental import pallas as pl

_SIZE = 128
_DOMAIN = 128
_BATCH = 1024
_ROWS = _BATCH * _SIZE
_BM = 1024


def _body(idx_ref, x_ref, o_ref):
    x = x_ref[:, :]
    m = jnp.max(x, axis=1, keepdims=True)
    e = jnp.exp(x - m)
    s = jnp.sum(e, axis=1, keepdims=True)
    lse = m + jnp.log(s)
    idx = idx_ref[0, :]
    d = lax.broadcasted_iota(jnp.int32, (_DOMAIN, _DOMAIN), 0)
    p = (d == idx[None, :]).astype(jnp.float32)
    g = lax.dot(x, p, precision=lax.Precision.HIGHEST,
                preferred_element_type=jnp.float32)
    o_ref[:, :] = g - lse


def kernel(inputs, log_probs):
    x = log_probs.reshape(_ROWS, _DOMAIN)
    idx = jnp.broadcast_to(inputs.astype(jnp.int32), (8, _SIZE))
    out = pl.pallas_call(
        _body,
        grid=(_ROWS // _BM,),
        in_specs=[
            pl.BlockSpec((8, _SIZE), lambda i: (0, 0)),
            pl.BlockSpec((_BM, _DOMAIN), lambda i: (i, 0)),
        ],
        out_specs=pl.BlockSpec((_BM, _DOMAIN), lambda i: (i, 0)),
        out_shape=jax.ShapeDtypeStruct((_ROWS, _DOMAIN), jnp.float32),
    )(idx, x)
    return out.reshape(_BATCH, _SIZE, _DOMAIN)


# speedup vs baseline: 2.0454x; 1.2027x over previous
import jax
import jax.numpy as jnp
from jax import lax
from jax.experimental import pallas as pl

_SIZE = 128
_DOMAIN = 128
_BATCH = 1024
_ROWS = _BATCH * _SIZE
_BM = 2048


def _body(idx_ref, x_ref, o_ref):
    x = x_ref[:, :]
    m = jnp.max(x, axis=1, keepdims=True)
    e = jnp.exp(x - m)
    s = jnp.sum(e, axis=1, keepdims=True)
    lse = m + jnp.log(s)
    idx = idx_ref[0, :]
    d = lax.broadcasted_iota(jnp.int32, (_DOMAIN, _DOMAIN), 0)
    p = (d == idx[None, :]).astype(jnp.float32)
    g = lax.dot(x, p, precision=lax.Precision.HIGHEST,
                preferred_element_type=jnp.float32)
    o_ref[:, :] = g - lse


def kernel(inputs, log_probs):
    x = log_probs.reshape(_ROWS, _DOMAIN)
    idx = jnp.broadcast_to(inputs.astype(jnp.int32), (8, _SIZE))
    out = pl.pallas_call(
        _body,
        grid=(_ROWS // _BM,),
        in_specs=[
            pl.BlockSpec((8, _SIZE), lambda i: (0, 0)),
            pl.BlockSpec((_BM, _DOMAIN), lambda i: (i, 0)),
        ],
        out_specs=pl.BlockSpec((_BM, _DOMAIN), lambda i: (i, 0)),
        out_shape=jax.ShapeDtypeStruct((_ROWS, _DOMAIN), jnp.float32),
    )(idx, x)
    return out.reshape(_BATCH, _SIZE, _DOMAIN)


# speedup vs baseline: 2.2702x; 1.1099x over previous
import jax
import jax.numpy as jnp
from jax import lax
from jax.experimental import pallas as pl

_SIZE = 128
_DOMAIN = 128
_BATCH = 1024
_ROWS = _BATCH * _SIZE
_BM = 4096


def _body(idx_ref, x_ref, o_ref):
    x = x_ref[:, :]
    m = jnp.max(x, axis=1, keepdims=True)
    e = jnp.exp(x - m)
    s = jnp.sum(e, axis=1, keepdims=True)
    lse = m + jnp.log(s)
    idx = idx_ref[0, :]
    d = lax.broadcasted_iota(jnp.int32, (_DOMAIN, _DOMAIN), 0)
    p = (d == idx[None, :]).astype(jnp.float32)
    g = lax.dot(x, p, precision=lax.Precision.HIGHEST,
                preferred_element_type=jnp.float32)
    o_ref[:, :] = g - lse


def kernel(inputs, log_probs):
    x = log_probs.reshape(_ROWS, _DOMAIN)
    idx = jnp.broadcast_to(inputs.astype(jnp.int32), (8, _SIZE))
    out = pl.pallas_call(
        _body,
        grid=(_ROWS // _BM,),
        in_specs=[
            pl.BlockSpec((8, _SIZE), lambda i: (0, 0)),
            pl.BlockSpec((_BM, _DOMAIN), lambda i: (i, 0)),
        ],
        out_specs=pl.BlockSpec((_BM, _DOMAIN), lambda i: (i, 0)),
        out_shape=jax.ShapeDtypeStruct((_ROWS, _DOMAIN), jnp.float32),
    )(idx, x)
    return out.reshape(_BATCH, _SIZE, _DOMAIN)


# speedup vs baseline: 2.3713x; 1.0445x over previous
import jax
import jax.numpy as jnp
from jax import lax
from jax.experimental import pallas as pl

_SIZE = 128
_DOMAIN = 128
_BATCH = 1024
_ROWS = _BATCH * _SIZE
_BM = 8192


def _body(idx_ref, x_ref, o_ref):
    x = x_ref[:, :]
    m = jnp.max(x, axis=1, keepdims=True)
    e = jnp.exp(x - m)
    s = jnp.sum(e, axis=1, keepdims=True)
    lse = m + jnp.log(s)
    idx = idx_ref[0, :]
    d = lax.broadcasted_iota(jnp.int32, (_DOMAIN, _DOMAIN), 0)
    p = (d == idx[None, :]).astype(jnp.float32)
    g = lax.dot(x, p, precision=lax.Precision.HIGHEST,
                preferred_element_type=jnp.float32)
    o_ref[:, :] = g - lse


def kernel(inputs, log_probs):
    x = log_probs.reshape(_ROWS, _DOMAIN)
    idx = jnp.broadcast_to(inputs.astype(jnp.int32), (8, _SIZE))
    out = pl.pallas_call(
        _body,
        grid=(_ROWS // _BM,),
        in_specs=[
            pl.BlockSpec((8, _SIZE), lambda i: (0, 0)),
            pl.BlockSpec((_BM, _DOMAIN), lambda i: (i, 0)),
        ],
        out_specs=pl.BlockSpec((_BM, _DOMAIN), lambda i: (i, 0)),
        out_shape=jax.ShapeDtypeStruct((_ROWS, _DOMAIN), jnp.float32),
    )(idx, x)
    return out.reshape(_BATCH, _SIZE, _DOMAIN)
